# Initial kernel scaffold; baseline (speedup 1.0000x reference)
import jax
import jax.numpy as jnp
from jax.experimental import pallas as pl


def kernel(x, W1, b1, Wc, bc):
    raise NotImplementedError("write your pallas kernel here")


# speedup vs baseline: 10.1564x; 10.1564x over previous
import jax
import jax.numpy as jnp
from jax.experimental import pallas as pl

_K = 100
_ALPHA = 0.5
_BB = 256


def _fused_body(x_ref, w1_ref, b1_ref, wc_ref, bc_ref, out_ref):
    f = jnp.dot(x_ref[...], w1_ref[...], preferred_element_type=jnp.float32)
    f = jnp.maximum(f + b1_ref[...], 0.0)

    bits = jax.lax.bitcast_convert_type(f, jnp.int32)
    rows = f.shape[0]
    lo = jnp.zeros((rows, 1), jnp.int32)
    hi = jnp.full((rows, 1), jnp.iinfo(jnp.int32).max, jnp.int32)

    def body(_, carry):
        lo, hi = carry
        mid = lo + (hi - lo) // 2
        cnt = jnp.sum((bits >= mid).astype(jnp.int32), axis=1, keepdims=True)
        take = cnt >= _K
        return jnp.where(take, mid, lo), jnp.where(take, hi, mid)

    lo, _ = jax.lax.fori_loop(0, 31, body, (lo, hi))

    scaled = jnp.where(bits >= lo, f, f * _ALPHA)
    out = jnp.dot(scaled, wc_ref[...], preferred_element_type=jnp.float32)
    out_ref[...] = out + bc_ref[...]


def kernel(x, W1, b1, Wc, bc):
    B, D_IN = x.shape
    D_FEAT = W1.shape[1]
    N = Wc.shape[1]
    N_PAD = ((N + 127) // 128) * 128
    Wc_p = jnp.pad(Wc, ((0, 0), (0, N_PAD - N)))
    bc_p = jnp.pad(bc, (0, N_PAD - N)).reshape(1, N_PAD)
    b1_r = b1.reshape(1, D_FEAT)

    out = pl.pallas_call(
        _fused_body,
        grid=(B // _BB,),
        in_specs=[
            pl.BlockSpec((_BB, D_IN), lambda i: (i, 0)),
            pl.BlockSpec((D_IN, D_FEAT), lambda i: (0, 0)),
            pl.BlockSpec((1, D_FEAT), lambda i: (0, 0)),
            pl.BlockSpec((D_FEAT, N_PAD), lambda i: (0, 0)),
            pl.BlockSpec((1, N_PAD), lambda i: (0, 0)),
        ],
        out_specs=pl.BlockSpec((_BB, N_PAD), lambda i: (i, 0)),
        out_shape=jax.ShapeDtypeStruct((B, N_PAD), jnp.float32),
    )(x, W1, b1_r, Wc_p, bc_p)
    return out[:, :N]
